# baseline (speedup 1.0000x reference)
import jax
import jax.numpy as jnp
from jax.experimental import pallas as pl
from jax.experimental.pallas import tpu as pltpu

NUM_CODES = 1024
DIM = 64
BLOCK_ROWS = 512
COMMIT_COST = 0.25


def _vq_block(z_ref, w_ref, zq_ref, idx_ref, part_ref):
    z = z_ref[...]
    w = w_ref[...]
    zsq = jnp.sum(z * z, axis=1, keepdims=True)
    wsq = jnp.sum(w * w, axis=1)
    zw = jax.lax.dot_general(
        z, w, (((1,), (1,)), ((), ())),
        precision=jax.lax.Precision.DEFAULT,
    )
    d = zsq - 2.0 * zw + wsq[None, :]
    m = jnp.min(d, axis=1, keepdims=True)
    col = jax.lax.broadcasted_iota(jnp.int32, d.shape, 1)
    idx = jnp.min(jnp.where(d == m, col, NUM_CODES), axis=1)
    oh = (col == idx[:, None]).astype(jnp.float32)
    zq = jax.lax.dot_general(
        oh, w, (((1,), (0,)), ((), ())),
        precision=jax.lax.Precision.HIGHEST,
    )
    zq_ref[...] = z + (zq - z)
    idx_ref[0, 0, :] = idx
    part_ref[0, 0, :] = jnp.full((128,), jnp.sum((zq - z) ** 2), jnp.float32)


def kernel(z, codebook):
    B, T, D = z.shape
    n = B * T
    nblocks = n // BLOCK_ROWS
    z_flat = z.reshape(n, D)
    zq_flat, idx, parts = pl.pallas_call(
        _vq_block,
        grid=(nblocks,),
        in_specs=[
            pl.BlockSpec((BLOCK_ROWS, D), lambda i: (i, 0)),
            pl.BlockSpec((NUM_CODES, D), lambda i: (0, 0)),
        ],
        out_specs=[
            pl.BlockSpec((BLOCK_ROWS, D), lambda i: (i, 0)),
            pl.BlockSpec((1, 1, BLOCK_ROWS), lambda i: (i, 0, 0)),
            pl.BlockSpec((1, 1, 128), lambda i: (i, 0, 0)),
        ],
        out_shape=[
            jax.ShapeDtypeStruct((n, D), jnp.float32),
            jax.ShapeDtypeStruct((nblocks, 1, BLOCK_ROWS), jnp.int32),
            jax.ShapeDtypeStruct((nblocks, 1, 128), jnp.float32),
        ],
        compiler_params=pltpu.CompilerParams(
            dimension_semantics=("parallel",),
        ),
    )(z_flat, codebook)
    loss = (1.0 + COMMIT_COST) * jnp.sum(parts[:, 0, 0]) / (n * D)
    return (
        zq_flat.reshape(B, T, D),
        loss,
        idx.reshape(B, T),
    )


# speedup vs baseline: 1.5979x; 1.5979x over previous
import jax
import jax.numpy as jnp
from jax.experimental import pallas as pl
from jax.experimental.pallas import tpu as pltpu

NUM_CODES = 1024
DIM = 64
BLOCK_ROWS = 512
COMMIT_COST = 0.25


def _vq_block(z_ref, w_ref, zq_ref, idx_ref, part_ref):
    z = z_ref[...]
    w = w_ref[...]
    zsq = jnp.sum(z * z, axis=1, keepdims=True)
    wsq = jnp.sum(w * w, axis=1)
    zw = jax.lax.dot_general(
        z, w, (((1,), (1,)), ((), ())),
        precision=jax.lax.Precision.DEFAULT,
    )
    d = zsq - 2.0 * zw + wsq[None, :]
    G = NUM_CODES // 128
    parts = [d[:, g * 128:(g + 1) * 128] for g in range(G)]
    pm = parts[0]
    for p in parts[1:]:
        pm = jnp.minimum(pm, p)
    m = jnp.min(pm, axis=1, keepdims=True)
    cola = jax.lax.broadcasted_iota(jnp.int32, pm.shape, 1)
    ci = jnp.where(parts[0] == m, cola, NUM_CODES)
    for g in range(1, G):
        cand = jnp.where(parts[g] == m, cola + g * 128, NUM_CODES)
        ci = jnp.minimum(ci, cand)
    idx = jnp.min(ci, axis=1)
    col = jax.lax.broadcasted_iota(jnp.int32, d.shape, 1)
    oh = (col == idx[:, None]).astype(jnp.float32)
    zq = jax.lax.dot_general(
        oh, w, (((1,), (0,)), ((), ())),
        precision=jax.lax.Precision.DEFAULT,
    )
    zq_ref[...] = z + (zq - z)
    idx_ref[0, 0, :] = idx
    part_ref[0, 0, :] = jnp.full((128,), jnp.sum((zq - z) ** 2), jnp.float32)


def kernel(z, codebook):
    B, T, D = z.shape
    n = B * T
    nblocks = n // BLOCK_ROWS
    z_flat = z.reshape(n, D)
    zq_flat, idx, parts = pl.pallas_call(
        _vq_block,
        grid=(nblocks,),
        in_specs=[
            pl.BlockSpec((BLOCK_ROWS, D), lambda i: (i, 0)),
            pl.BlockSpec((NUM_CODES, D), lambda i: (0, 0)),
        ],
        out_specs=[
            pl.BlockSpec((BLOCK_ROWS, D), lambda i: (i, 0)),
            pl.BlockSpec((1, 1, BLOCK_ROWS), lambda i: (i, 0, 0)),
            pl.BlockSpec((1, 1, 128), lambda i: (i, 0, 0)),
        ],
        out_shape=[
            jax.ShapeDtypeStruct((n, D), jnp.float32),
            jax.ShapeDtypeStruct((nblocks, 1, BLOCK_ROWS), jnp.int32),
            jax.ShapeDtypeStruct((nblocks, 1, 128), jnp.float32),
        ],
        compiler_params=pltpu.CompilerParams(
            dimension_semantics=("parallel",),
        ),
    )(z_flat, codebook)
    loss = (1.0 + COMMIT_COST) * jnp.sum(parts[:, 0, 0]) / (n * D)
    return (
        zq_flat.reshape(B, T, D),
        loss,
        idx.reshape(B, T),
    )


# speedup vs baseline: 1.7424x; 1.0904x over previous
import jax
import jax.numpy as jnp
from jax.experimental import pallas as pl
from jax.experimental.pallas import tpu as pltpu

NUM_CODES = 1024
DIM = 64
BLOCK_ROWS = 512
COMMIT_COST = 0.25


def _vq_block(z_ref, w_ref, zq_ref, idx_ref, part_ref):
    z = z_ref[...]
    w = w_ref[...]
    zsq = jnp.sum(z * z, axis=1, keepdims=True)
    wsq = jnp.sum(w * w, axis=1)
    zw = jax.lax.dot_general(
        z, w, (((1,), (1,)), ((), ())),
        precision=jax.lax.Precision.DEFAULT,
    )
    G = NUM_CODES // 128
    pm = zsq - 2.0 * zw[:, 0:128] + wsq[None, 0:128]
    gidx = jnp.zeros(pm.shape, jnp.int32)
    for g in range(1, G):
        dg = zsq - 2.0 * zw[:, g * 128:(g + 1) * 128] + wsq[None, g * 128:(g + 1) * 128]
        mask = dg < pm
        pm = jnp.where(mask, dg, pm)
        gidx = jnp.where(mask, g, gidx)
    m = jnp.min(pm, axis=1, keepdims=True)
    lane = jax.lax.broadcasted_iota(jnp.int32, pm.shape, 1)
    ci = jnp.where(pm == m, gidx * 128 + lane, NUM_CODES)
    idx = jnp.min(ci, axis=1)
    col = jax.lax.broadcasted_iota(jnp.int32, zw.shape, 1)
    oh = (col == idx[:, None]).astype(jnp.float32)
    zq = jax.lax.dot_general(
        oh, w, (((1,), (0,)), ((), ())),
        precision=jax.lax.Precision.DEFAULT,
    )
    zq_ref[...] = z + (zq - z)
    idx_ref[...] = idx
    part_ref[...] = jnp.full((128,), jnp.sum((zq - z) ** 2), jnp.float32)


def kernel(z, codebook):
    B, T, D = z.shape
    n = B * T
    nblocks = n // BLOCK_ROWS
    z_flat = z.reshape(n, D)
    zq_flat, idx, parts = pl.pallas_call(
        _vq_block,
        grid=(nblocks,),
        in_specs=[
            pl.BlockSpec((BLOCK_ROWS, D), lambda i: (i, 0)),
            pl.BlockSpec((NUM_CODES, D), lambda i: (0, 0)),
        ],
        out_specs=[
            pl.BlockSpec((BLOCK_ROWS, D), lambda i: (i, 0)),
            pl.BlockSpec((BLOCK_ROWS,), lambda i: (i,)),
            pl.BlockSpec((128,), lambda i: (i,)),
        ],
        out_shape=[
            jax.ShapeDtypeStruct((n, D), jnp.float32),
            jax.ShapeDtypeStruct((n,), jnp.int32),
            jax.ShapeDtypeStruct((nblocks * 128,), jnp.float32),
        ],
        compiler_params=pltpu.CompilerParams(
            dimension_semantics=("parallel",),
        ),
    )(z_flat, codebook)
    loss = (1.0 + COMMIT_COST) * jnp.sum(parts) / (128.0 * n * D)
    return (
        zq_flat.reshape(B, T, D),
        loss,
        idx.reshape(B, T),
    )


# speedup vs baseline: 1.8519x; 1.0629x over previous
import jax
import jax.numpy as jnp
from jax.experimental import pallas as pl
from jax.experimental.pallas import tpu as pltpu

NUM_CODES = 1024
DIM = 64
COMMIT_COST = 0.25
ROWS_PER_BATCH = 576
IDX_TILE = 8


def _vq_block(z_ref, w_ref, zq_ref, idx_ref, part_ref):
    z = z_ref[0]
    w = w_ref[...]
    zsq = jnp.sum(z * z, axis=1, keepdims=True)
    wsq = jnp.sum(w * w, axis=1)
    zw = jax.lax.dot_general(
        z, w, (((1,), (1,)), ((), ())),
        precision=jax.lax.Precision.DEFAULT,
    )
    G = NUM_CODES // 128
    pm = zsq - 2.0 * zw[:, 0:128] + wsq[None, 0:128]
    gidx = jnp.zeros(pm.shape, jnp.int32)
    for g in range(1, G):
        dg = zsq - 2.0 * zw[:, g * 128:(g + 1) * 128] + wsq[None, g * 128:(g + 1) * 128]
        mask = dg < pm
        pm = jnp.where(mask, dg, pm)
        gidx = jnp.where(mask, g, gidx)
    m = jnp.min(pm, axis=1, keepdims=True)
    lane = jax.lax.broadcasted_iota(jnp.int32, pm.shape, 1)
    ci = jnp.where(pm == m, gidx * 128 + lane, NUM_CODES)
    idx = jnp.min(ci, axis=1)
    col = jax.lax.broadcasted_iota(jnp.int32, zw.shape, 1)
    oh = (col == idx[:, None]).astype(jnp.float32)
    zq = jax.lax.dot_general(
        oh, w, (((1,), (0,)), ((), ())),
        precision=jax.lax.Precision.DEFAULT,
    )
    zq_ref[0] = z + (zq - z)
    idx_ref[pl.program_id(0) % IDX_TILE, :] = idx
    part_ref[...] = jnp.full((128,), jnp.sum((zq - z) ** 2), jnp.float32)


def kernel(z, codebook):
    B, T, D = z.shape
    n = B * T
    zq, idx, parts = pl.pallas_call(
        _vq_block,
        grid=(B,),
        in_specs=[
            pl.BlockSpec((1, T, D), lambda i: (i, 0, 0)),
            pl.BlockSpec((NUM_CODES, D), lambda i: (0, 0)),
        ],
        out_specs=[
            pl.BlockSpec((1, T, D), lambda i: (i, 0, 0)),
            pl.BlockSpec((IDX_TILE, T), lambda i: (i // IDX_TILE, 0)),
            pl.BlockSpec((128,), lambda i: (i,)),
        ],
        out_shape=[
            jax.ShapeDtypeStruct((B, T, D), jnp.float32),
            jax.ShapeDtypeStruct((B, T), jnp.int32),
            jax.ShapeDtypeStruct((B * 128,), jnp.float32),
        ],
        compiler_params=pltpu.CompilerParams(
            dimension_semantics=("arbitrary",),
        ),
    )(z, codebook)
    loss = (1.0 + COMMIT_COST) * jnp.sum(parts) / (128.0 * n * D)
    return (zq, loss, idx)


# speedup vs baseline: 2.0882x; 1.1276x over previous
import jax
import jax.numpy as jnp
from jax.experimental import pallas as pl
from jax.experimental.pallas import tpu as pltpu

NUM_CODES = 1024
DIM = 64
COMMIT_COST = 0.25
BATCHES_PER_STEP = 2
IDX_TILE = 8


def _vq_block(z_ref, w_ref, zq_ref, idx_ref, part_ref):
    w = w_ref[...]
    wsq = jnp.sum(w * w, axis=1)
    ones_b = jnp.ones((DIM, 128), jnp.float32)
    G = NUM_CODES // 128
    total = jnp.zeros((), jnp.float32)
    for b in range(BATCHES_PER_STEP):
        z = z_ref[b]
        zw2 = jax.lax.dot_general(
            z * (-2.0), w, (((1,), (1,)), ((), ())),
            precision=jax.lax.Precision.DEFAULT,
        )
        zsqb = jax.lax.dot_general(
            z * z, ones_b, (((1,), (0,)), ((), ())),
            precision=jax.lax.Precision.DEFAULT,
        )
        pm = zsqb + zw2[:, 0:128] + wsq[None, 0:128]
        gidx = jnp.zeros(pm.shape, jnp.int32)
        for g in range(1, G):
            dg = zsqb + zw2[:, g * 128:(g + 1) * 128] + wsq[None, g * 128:(g + 1) * 128]
            mask = dg < pm
            pm = jnp.where(mask, dg, pm)
            gidx = jnp.where(mask, g, gidx)
        m = jnp.min(pm, axis=1, keepdims=True)
        lane = jax.lax.broadcasted_iota(jnp.int32, pm.shape, 1)
        ci = jnp.where(pm == m, gidx * 128 + lane, NUM_CODES)
        idx = jnp.min(ci, axis=1)
        col = jax.lax.broadcasted_iota(jnp.int32, zw2.shape, 1)
        oh = (col == idx[:, None]).astype(jnp.float32)
        zq = jax.lax.dot_general(
            oh, w, (((1,), (0,)), ((), ())),
            precision=jax.lax.Precision.DEFAULT,
        )
        zq_ref[b] = z + (zq - z)
        row = pl.program_id(0) % (IDX_TILE // BATCHES_PER_STEP)
        idx_ref[row * BATCHES_PER_STEP + b, :] = idx
        total = total + jnp.sum((zq - z) ** 2)
    part_ref[...] = jnp.full((128,), total, jnp.float32)


def kernel(z, codebook):
    B, T, D = z.shape
    n = B * T
    nsteps = B // BATCHES_PER_STEP
    zq, idx, parts = pl.pallas_call(
        _vq_block,
        grid=(nsteps,),
        in_specs=[
            pl.BlockSpec((BATCHES_PER_STEP, T, D), lambda i: (i, 0, 0)),
            pl.BlockSpec((NUM_CODES, D), lambda i: (0, 0)),
        ],
        out_specs=[
            pl.BlockSpec((BATCHES_PER_STEP, T, D), lambda i: (i, 0, 0)),
            pl.BlockSpec(
                (IDX_TILE, T),
                lambda i: (i // (IDX_TILE // BATCHES_PER_STEP), 0),
            ),
            pl.BlockSpec((128,), lambda i: (i,)),
        ],
        out_shape=[
            jax.ShapeDtypeStruct((B, T, D), jnp.float32),
            jax.ShapeDtypeStruct((B, T), jnp.int32),
            jax.ShapeDtypeStruct((nsteps * 128,), jnp.float32),
        ],
        compiler_params=pltpu.CompilerParams(
            dimension_semantics=("arbitrary",),
        ),
    )(z, codebook)
    loss = (1.0 + COMMIT_COST) * jnp.sum(parts) / (128.0 * n * D)
    return (zq, loss, idx)


# speedup vs baseline: 2.2484x; 1.0767x over previous
import jax
import jax.numpy as jnp
from jax.experimental import pallas as pl
from jax.experimental.pallas import tpu as pltpu

NUM_CODES = 1024
DIM = 64
COMMIT_COST = 0.25
BATCHES_PER_STEP = 4
IDX_TILE = 8


def _vq_block(z_ref, w_ref, zq_ref, idx_ref, part_ref):
    w = w_ref[...]
    wsq = jnp.sum(w * w, axis=1)
    ones_b = jnp.ones((DIM, 128), jnp.float32)
    G = NUM_CODES // 128
    total = jnp.zeros((), jnp.float32)
    for b in range(BATCHES_PER_STEP):
        z = z_ref[b]
        zw2 = jax.lax.dot_general(
            z * (-2.0), w, (((1,), (1,)), ((), ())),
            precision=jax.lax.Precision.DEFAULT,
        )
        zsqb = jax.lax.dot_general(
            z * z, ones_b, (((1,), (0,)), ((), ())),
            precision=jax.lax.Precision.DEFAULT,
        )
        pm = zsqb + zw2[:, 0:128] + wsq[None, 0:128]
        gidx = jnp.zeros(pm.shape, jnp.int32)
        for g in range(1, G):
            dg = zsqb + zw2[:, g * 128:(g + 1) * 128] + wsq[None, g * 128:(g + 1) * 128]
            mask = dg < pm
            pm = jnp.where(mask, dg, pm)
            gidx = jnp.where(mask, g, gidx)
        m = jnp.min(pm, axis=1, keepdims=True)
        lane = jax.lax.broadcasted_iota(jnp.int32, pm.shape, 1)
        ci = jnp.where(pm == m, gidx * 128 + lane, NUM_CODES)
        idx = jnp.min(ci, axis=1)
        col = jax.lax.broadcasted_iota(jnp.int32, zw2.shape, 1)
        oh = (col == idx[:, None]).astype(jnp.float32)
        zq = jax.lax.dot_general(
            oh, w, (((1,), (0,)), ((), ())),
            precision=jax.lax.Precision.DEFAULT,
        )
        zq_ref[b] = z + (zq - z)
        row = pl.program_id(0) % (IDX_TILE // BATCHES_PER_STEP)
        idx_ref[row * BATCHES_PER_STEP + b, :] = idx
        total = total + jnp.sum((zq - z) ** 2)
    part_ref[...] = jnp.full((128,), total, jnp.float32)


def kernel(z, codebook):
    B, T, D = z.shape
    n = B * T
    nsteps = B // BATCHES_PER_STEP
    zq, idx, parts = pl.pallas_call(
        _vq_block,
        grid=(nsteps,),
        in_specs=[
            pl.BlockSpec((BATCHES_PER_STEP, T, D), lambda i: (i, 0, 0)),
            pl.BlockSpec((NUM_CODES, D), lambda i: (0, 0)),
        ],
        out_specs=[
            pl.BlockSpec((BATCHES_PER_STEP, T, D), lambda i: (i, 0, 0)),
            pl.BlockSpec(
                (IDX_TILE, T),
                lambda i: (i // (IDX_TILE // BATCHES_PER_STEP), 0),
            ),
            pl.BlockSpec((128,), lambda i: (i,)),
        ],
        out_shape=[
            jax.ShapeDtypeStruct((B, T, D), jnp.float32),
            jax.ShapeDtypeStruct((B, T), jnp.int32),
            jax.ShapeDtypeStruct((nsteps * 128,), jnp.float32),
        ],
        compiler_params=pltpu.CompilerParams(
            dimension_semantics=("arbitrary",),
        ),
    )(z, codebook)
    loss = (1.0 + COMMIT_COST) * jnp.sum(parts) / (128.0 * n * D)
    return (zq, loss, idx)
